# baseline (speedup 1.0000x reference)
import functools

import jax
import jax.numpy as jnp
from jax.experimental import pallas as pl

H, D = 8, 16
NODE_TYPES = ["drug", "protein"]
EDGE_TYPES = [("drug", "dd", "drug"), ("protein", "pd", "drug"),
              ("drug", "dp", "protein"), ("protein", "pp", "protein")]
RELS = ["dd", "pd", "dp", "pp"]


def _mm_bias_kernel(x_ref, w_ref, b_ref, o_ref):
    o_ref[...] = jnp.dot(x_ref[...], w_ref[...],
                         preferred_element_type=jnp.float32) + b_ref[...]


@functools.partial(jax.jit, static_argnames=("block",))
def _mm_bias(x, w, b, block=1000):
    n, k = x.shape
    m = w.shape[1]
    pad = (-n) % block
    if pad:
        x = jnp.concatenate([x, jnp.zeros((pad, k), x.dtype)], axis=0)
    np_ = x.shape[0]
    y = pl.pallas_call(
        _mm_bias_kernel,
        grid=(np_ // block,),
        in_specs=[
            pl.BlockSpec((block, k), lambda i: (i, 0)),
            pl.BlockSpec((k, m), lambda i: (0, 0)),
            pl.BlockSpec((1, m), lambda i: (0, 0)),
        ],
        out_specs=pl.BlockSpec((block, m), lambda i: (i, 0)),
        out_shape=jax.ShapeDtypeStruct((np_, m), jnp.float32),
    )(x, w, b.reshape(1, m))
    return y[:n] if pad else y


def _han_conv(x_dict, ei_dict, P, l):
    hid = P[f"p{l}_W_drug"].shape[0]
    xp = {t: _mm_bias(x_dict[t], P[f"p{l}_W_{t}"], P[f"p{l}_b_{t}"])
          .reshape(-1, H, D) for t in NODE_TYPES}
    out_dict = {t: [] for t in NODE_TYPES}
    for (src_t, rel, dst_t) in EDGE_TYPES:
        ei = ei_dict[rel]
        x_src, x_dst = xp[src_t], xp[dst_t]
        a_src = (x_src * P[f"a{l}_src_{rel}"]).sum(-1)
        a_dst = (x_dst * P[f"a{l}_dst_{rel}"]).sum(-1)
        s, d = ei[0], ei[1]
        alpha = jax.nn.leaky_relu(a_src[s] + a_dst[d], 0.2)
        n_dst = x_dst.shape[0]
        amax = jax.ops.segment_max(alpha, d, num_segments=n_dst)
        amax = jnp.where(jnp.isfinite(amax), amax, 0.0)
        ex = jnp.exp(alpha - amax[d])
        denom = jax.ops.segment_sum(ex, d, num_segments=n_dst)
        w = ex / (denom[d] + 1e-16)
        msg = x_src[s] * w[..., None]
        agg = jax.ops.segment_sum(msg, d, num_segments=n_dst)
        out_dict[dst_t].append(jax.nn.relu(agg).reshape(n_dst, H * D))
    res = {}
    for t in NODE_TYPES:
        st = jnp.stack(out_dict[t])
        r, n, f = st.shape
        kmat = _mm_bias(st.reshape(r * n, f), P[f"k{l}_W"], P[f"k{l}_b"])
        kvec = jnp.tanh(kmat).reshape(r, n, f).mean(axis=1)
        score = (P[f"q{l}"] * kvec).sum(-1)
        attn = jax.nn.softmax(score)
        res[t] = (attn[:, None, None] * st).sum(0)
    return res


def kernel(x_drug, x_protein, edge_index_dd, edge_index_pd, edge_index_dp,
           edge_index_pp, p0_W_drug, p0_b_drug, p0_W_protein, p0_b_protein,
           a0_src_dd, a0_dst_dd, a0_src_pd, a0_dst_pd, a0_src_dp, a0_dst_dp,
           a0_src_pp, a0_dst_pp, k0_W, k0_b, q0, p1_W_drug, p1_b_drug,
           p1_W_protein, p1_b_protein, a1_src_dd, a1_dst_dd, a1_src_pd,
           a1_dst_pd, a1_src_dp, a1_dst_dp, a1_src_pp, a1_dst_pp, k1_W, k1_b,
           q1, lin_W, lin_b):
    kw = dict(locals())
    ei_dict = {r: kw["edge_index_" + r] for r in RELS}
    x_dict = {"drug": kw["x_drug"], "protein": kw["x_protein"]}
    out = _han_conv(x_dict, ei_dict, kw, 0)
    out = _han_conv(out, ei_dict, kw, 1)
    return _mm_bias(out["drug"], kw["lin_W"], kw["lin_b"])
